# baseline (speedup 1.0000x reference)
import functools

import jax
import jax.numpy as jnp
from jax import lax
from jax.experimental import pallas as pl
from jax.experimental.pallas import tpu as pltpu
from jax.experimental.pallas import tpu_sc as plsc

B = 16384
EMB = 32
PACK = 4
LANES = EMB * PACK
NC = 2
NS = 16
NW = NC * NS
BPW = B // NW
CH = 128
NCHUNK = BPW // CH
NSLOT = 4
NCH_TOT = 2 * NCHUNK


def _gather_body(user_tab, store_tab, uid, sid, ug_out, sg_out,
                 uidx_v, sidx_v, buf0, buf1, buf2, buf3, gsem, wsem):
    bufs = [buf0, buf1, buf2, buf3]
    wid = lax.axis_index("s") * NC + lax.axis_index("c")
    base = wid * BPW
    pltpu.sync_copy(uid.at[wid], uidx_v)
    pltpu.sync_copy(sid.at[wid], sidx_v)
    for j in range(NCHUNK):
        for k in range(CH // 16):
            s = pl.ds(k * 16, 16)
            uidx_v[j, s] = lax.shift_right_logical(uidx_v[j, s], 2)
            sidx_v[j, s] = lax.shift_right_logical(sidx_v[j, s], 2)

    def chunk_src(ci):
        t, j = ci % 2, ci // 2
        idxv = uidx_v if t == 0 else sidx_v
        tab = user_tab if t == 0 else store_tab
        return tab.at[idxv.at[j]]

    def chunk_dst(ci):
        t, j = ci % 2, ci // 2
        out = ug_out if t == 0 else sg_out
        return out.at[pl.ds(base + j * CH, CH)]

    gh = {}
    wh = {}
    for ci in range(NCH_TOT + 3):
        if ci < NCH_TOT:
            if ci >= NSLOT:
                wh[ci - NSLOT].wait()
            gh[ci] = pltpu.async_copy(chunk_src(ci), bufs[ci % NSLOT], gsem)
        wi = ci - 3
        if 0 <= wi < NCH_TOT:
            gh[wi].wait()
            wh[wi] = pltpu.async_copy(bufs[wi % NSLOT], chunk_dst(wi), wsem)
    for wi in range(NCH_TOT - 3, NCH_TOT):
        wh[wi].wait()


_sc_gather = pl.kernel(
    _gather_body,
    out_type=(
        jax.ShapeDtypeStruct((B, LANES), jnp.float32),
        jax.ShapeDtypeStruct((B, LANES), jnp.float32),
    ),
    mesh=plsc.VectorSubcoreMesh(core_axis_name="c", subcore_axis_name="s"),
    scratch_types=[
        pltpu.VMEM((NCHUNK, CH), jnp.int32),
        pltpu.VMEM((NCHUNK, CH), jnp.int32),
        pltpu.VMEM((CH, LANES), jnp.float32),
        pltpu.VMEM((CH, LANES), jnp.float32),
        pltpu.VMEM((CH, LANES), jnp.float32),
        pltpu.VMEM((CH, LANES), jnp.float32),
        pltpu.SemaphoreType.DMA,
        pltpu.SemaphoreType.DMA,
    ],
)

BLK = 2048


def _mlp_body(uid, sid, ug, sg, f, w1u4, w1s4, w1f, b1, w2, b2, w3, b3, out):
    lane = lax.broadcasted_iota(jnp.int32, (BLK, LANES), 1) // EMB
    um = (lane == (uid[...] & 3)).astype(jnp.float32)
    sm = (lane == (sid[...] & 3)).astype(jnp.float32)
    h = jnp.dot(ug[...] * um, w1u4[...], preferred_element_type=jnp.float32)
    h += jnp.dot(sg[...] * sm, w1s4[...], preferred_element_type=jnp.float32)
    h += jnp.dot(f[...], w1f[...], preferred_element_type=jnp.float32)
    h = jnp.maximum(h + b1[...], 0.0)
    h2 = jnp.dot(h, w2[...], preferred_element_type=jnp.float32)
    h2 = jnp.maximum(h2 + b2[...], 0.0)
    o = jnp.dot(h2, w3[...], preferred_element_type=jnp.float32) + b3[...]
    out[...] = 1.0 / (1.0 + jnp.exp(-o))


def _rows(i):
    return (i, 0)


@jax.jit
def kernel(user_id, store_id, sentiment, rating, distance, hour_sin,
           user_table, store_table, W1, b1, W2, b2, W3, b3):
    uid = user_id.astype(jnp.int32)
    sid = store_id.astype(jnp.int32)
    ut4 = user_table.reshape(-1, LANES)
    st4 = store_table.reshape(-1, LANES)
    ug, sg = _sc_gather(ut4, st4,
                        uid.reshape(NW, NCHUNK, CH), sid.reshape(NW, NCHUNK, CH))

    f = jnp.stack([sentiment, rating, distance, hour_sin], axis=1)
    w1u4 = jnp.concatenate([W1[:EMB]] * PACK, axis=0)
    w1s4 = jnp.concatenate([W1[EMB:2 * EMB]] * PACK, axis=0)
    w1f = W1[2 * EMB:]

    full = lambda shape: pl.BlockSpec(shape, lambda i: (0, 0))
    out = pl.pallas_call(
        _mlp_body,
        grid=(B // BLK,),
        in_specs=[
            pl.BlockSpec((BLK, 1), _rows),
            pl.BlockSpec((BLK, 1), _rows),
            pl.BlockSpec((BLK, LANES), _rows),
            pl.BlockSpec((BLK, LANES), _rows),
            pl.BlockSpec((BLK, 4), _rows),
            full((LANES, 64)),
            full((LANES, 64)),
            full((4, 64)),
            full((1, 64)),
            full((64, 32)),
            full((1, 32)),
            full((32, 1)),
            full((1, 1)),
        ],
        out_specs=pl.BlockSpec((BLK, 1), _rows),
        out_shape=jax.ShapeDtypeStruct((B, 1), jnp.float32),
    )(uid.reshape(B, 1), sid.reshape(B, 1), ug, sg, f, w1u4, w1s4, w1f,
      b1.reshape(1, 64), W2, b2.reshape(1, 32), W3, b3.reshape(1, 1))
    return out.reshape(B)


# speedup vs baseline: 1.0091x; 1.0091x over previous
import functools

import jax
import jax.numpy as jnp
from jax import lax
from jax.experimental import pallas as pl
from jax.experimental.pallas import tpu as pltpu
from jax.experimental.pallas import tpu_sc as plsc

B = 16384
EMB = 32
PACK = 4
LANES = EMB * PACK
NC = 2
NS = 16
NW = NC * NS
BPW = B // NW
CH = 128
NCHUNK = BPW // CH
NSLOT = 4
NCH_TOT = 2 * NCHUNK


def _gather_body(user_tab, store_tab, uid, sid, ug_out, sg_out,
                 uidx_v, sidx_v, buf0, buf1, buf2, buf3, gsem, wsem):
    bufs = [buf0, buf1, buf2, buf3]
    wid = lax.axis_index("s") * NC + lax.axis_index("c")
    base = wid * BPW
    pltpu.sync_copy(uid.at[wid], uidx_v)
    pltpu.sync_copy(sid.at[wid], sidx_v)
    for j in range(NCHUNK):
        for k in range(CH // 16):
            s = pl.ds(k * 16, 16)
            uidx_v[j, s] = lax.shift_right_logical(uidx_v[j, s], 2)
            sidx_v[j, s] = lax.shift_right_logical(sidx_v[j, s], 2)

    def chunk_src(ci):
        t, j = ci % 2, ci // 2
        idxv = uidx_v if t == 0 else sidx_v
        tab = user_tab if t == 0 else store_tab
        return tab.at[idxv.at[j]]

    def chunk_dst(ci):
        t, j = ci % 2, ci // 2
        out = ug_out if t == 0 else sg_out
        return out.at[pl.ds(base + j * CH, CH)]

    gh = {}
    wh = {}
    for ci in range(NCH_TOT + 3):
        if ci < NCH_TOT:
            if ci >= NSLOT:
                wh[ci - NSLOT].wait()
            gh[ci] = pltpu.async_copy(chunk_src(ci), bufs[ci % NSLOT], gsem)
        wi = ci - 3
        if 0 <= wi < NCH_TOT:
            gh[wi].wait()
            wh[wi] = pltpu.async_copy(bufs[wi % NSLOT], chunk_dst(wi), wsem)
    for wi in range(NCH_TOT - 3, NCH_TOT):
        wh[wi].wait()


_sc_gather = pl.kernel(
    _gather_body,
    out_type=(
        jax.ShapeDtypeStruct((B, LANES), jnp.float32),
        jax.ShapeDtypeStruct((B, LANES), jnp.float32),
    ),
    mesh=plsc.VectorSubcoreMesh(core_axis_name="c", subcore_axis_name="s"),
    scratch_types=[
        pltpu.VMEM((NCHUNK, CH), jnp.int32),
        pltpu.VMEM((NCHUNK, CH), jnp.int32),
        pltpu.VMEM((CH, LANES), jnp.float32),
        pltpu.VMEM((CH, LANES), jnp.float32),
        pltpu.VMEM((CH, LANES), jnp.float32),
        pltpu.VMEM((CH, LANES), jnp.float32),
        pltpu.SemaphoreType.DMA,
        pltpu.SemaphoreType.DMA,
    ],
)

BLK = 2048


def _mlp_body(uid, sid, ug, sg, f, w1u4, w1s4, w1f, b1, w2, b2, w3t, b3, out):
    lane = lax.broadcasted_iota(jnp.int32, (BLK, LANES), 1) // EMB
    um = (lane == (uid[...] & 3)).astype(jnp.float32)
    sm = (lane == (sid[...] & 3)).astype(jnp.float32)
    h = jnp.dot(ug[...] * um, w1u4[...], preferred_element_type=jnp.float32)
    h += jnp.dot(sg[...] * sm, w1s4[...], preferred_element_type=jnp.float32)
    h += jnp.dot(f[...], w1f[...], preferred_element_type=jnp.float32)
    h = jnp.maximum(h + b1[...], 0.0)
    h2 = jnp.dot(h, w2[...], preferred_element_type=jnp.float32)
    h2 = jnp.maximum(h2 + b2[...], 0.0)
    o = jnp.sum(h2 * w3t[...], axis=1) + b3[0, 0]
    out[...] = 1.0 / (1.0 + jnp.exp(-o))


def _rows(i):
    return (i, 0)


@jax.jit
def kernel(user_id, store_id, sentiment, rating, distance, hour_sin,
           user_table, store_table, W1, b1, W2, b2, W3, b3):
    uid = user_id.astype(jnp.int32)
    sid = store_id.astype(jnp.int32)
    ut4 = user_table.reshape(-1, LANES)
    st4 = store_table.reshape(-1, LANES)
    ug, sg = _sc_gather(ut4, st4,
                        uid.reshape(NW, NCHUNK, CH), sid.reshape(NW, NCHUNK, CH))

    f = jnp.stack([sentiment, rating, distance, hour_sin], axis=1)
    w1u4 = jnp.concatenate([W1[:EMB]] * PACK, axis=0)
    w1s4 = jnp.concatenate([W1[EMB:2 * EMB]] * PACK, axis=0)
    w1f = W1[2 * EMB:]

    full = lambda shape: pl.BlockSpec(shape, lambda i: (0, 0))
    out = pl.pallas_call(
        _mlp_body,
        grid=(B // BLK,),
        in_specs=[
            pl.BlockSpec((BLK, 1), _rows),
            pl.BlockSpec((BLK, 1), _rows),
            pl.BlockSpec((BLK, LANES), _rows),
            pl.BlockSpec((BLK, LANES), _rows),
            pl.BlockSpec((BLK, 4), _rows),
            full((LANES, 64)),
            full((LANES, 64)),
            full((4, 64)),
            full((1, 64)),
            full((64, 32)),
            full((1, 32)),
            full((1, 32)),
            full((1, 1)),
        ],
        out_specs=pl.BlockSpec((BLK,), lambda i: (i,)),
        out_shape=jax.ShapeDtypeStruct((B,), jnp.float32),
    )(uid.reshape(B, 1), sid.reshape(B, 1), ug, sg, f, w1u4, w1s4, w1f,
      b1.reshape(1, 64), W2, b2.reshape(1, 32), W3.reshape(1, 32),
      b3.reshape(1, 1))
    return out


# speedup vs baseline: 1.4988x; 1.4852x over previous
import functools

import jax
import jax.numpy as jnp
from jax import lax
from jax.experimental import pallas as pl
from jax.experimental.pallas import tpu as pltpu
from jax.experimental.pallas import tpu_sc as plsc

B = 16384
EMB = 32
PACK = 4
LANES = EMB * PACK
NU = 1000000
NST = 100000
TBLK = 1024
S_U = 250880
S_S = 25600
NC = 2
NS = 16
NW = NC * NS
BPW = B // NW
CH = 128
NCHUNK = BPW // CH
NSLOT = 4
NCH_TOT = 2 * NCHUNK


def _repack_body(x0, x1, x2, x3, eye, out):
    for k, xb in enumerate((x0, x1, x2, x3)):
        y = lax.dot_general(xb[...], eye[...], (((0,), (0,)), ((), ())),
                            preferred_element_type=jnp.float32)
        out[:, k * EMB:(k + 1) * EMB] = y


def _repack(t32, n_rows, stride):
    nb = stride // TBLK
    nb_max = -(-n_rows // TBLK) - 1

    def spec(k):
        return pl.BlockSpec(
            (EMB, TBLK), lambda i, k=k: (0, jnp.minimum(i + nb * k, nb_max)))

    eye = jnp.eye(EMB, dtype=jnp.float32)
    return pl.pallas_call(
        _repack_body,
        grid=(nb,),
        in_specs=[spec(0), spec(1), spec(2), spec(3),
                  pl.BlockSpec((EMB, EMB), lambda i: (0, 0))],
        out_specs=pl.BlockSpec((TBLK, LANES), lambda i: (i, 0)),
        out_shape=jax.ShapeDtypeStruct((stride, LANES), jnp.float32),
    )(t32, t32, t32, t32, eye)


def _section_and_row(v, stride):
    one = jnp.int32(1)
    zero = jnp.int32(0)
    k = jnp.where(v >= stride, one, zero)
    k += jnp.where(v >= 2 * stride, one, zero)
    k += jnp.where(v >= 3 * stride, one, zero)
    return k, v - k * jnp.int32(stride)


def _gather_body(user_tab, store_tab, uid, sid, ug_out, sg_out,
                 uidx_v, sidx_v, buf0, buf1, buf2, buf3, gsem, wsem):
    bufs = [buf0, buf1, buf2, buf3]
    wid = lax.axis_index("s") * NC + lax.axis_index("c")
    base = wid * BPW
    pltpu.sync_copy(uid.at[wid], uidx_v)
    pltpu.sync_copy(sid.at[wid], sidx_v)
    for j in range(NCHUNK):
        for t in range(CH // 16):
            s = pl.ds(t * 16, 16)
            _, qu = _section_and_row(uidx_v[j, s], S_U)
            uidx_v[j, s] = qu
            _, qs = _section_and_row(sidx_v[j, s], S_S)
            sidx_v[j, s] = qs

    def chunk_src(ci):
        t, j = ci % 2, ci // 2
        idxv = uidx_v if t == 0 else sidx_v
        tab = user_tab if t == 0 else store_tab
        return tab.at[idxv.at[j]]

    def chunk_dst(ci):
        t, j = ci % 2, ci // 2
        out = ug_out if t == 0 else sg_out
        return out.at[pl.ds(base + j * CH, CH)]

    gh = {}
    wh = {}
    for ci in range(NCH_TOT + 3):
        if ci < NCH_TOT:
            if ci >= NSLOT:
                wh[ci - NSLOT].wait()
            gh[ci] = pltpu.async_copy(chunk_src(ci), bufs[ci % NSLOT], gsem)
        wi = ci - 3
        if 0 <= wi < NCH_TOT:
            gh[wi].wait()
            wh[wi] = pltpu.async_copy(bufs[wi % NSLOT], chunk_dst(wi), wsem)
    for wi in range(NCH_TOT - 3, NCH_TOT):
        wh[wi].wait()


_sc_gather = pl.kernel(
    _gather_body,
    out_type=(
        jax.ShapeDtypeStruct((B, LANES), jnp.float32),
        jax.ShapeDtypeStruct((B, LANES), jnp.float32),
    ),
    mesh=plsc.VectorSubcoreMesh(core_axis_name="c", subcore_axis_name="s"),
    scratch_types=[
        pltpu.VMEM((NCHUNK, CH), jnp.int32),
        pltpu.VMEM((NCHUNK, CH), jnp.int32),
        pltpu.VMEM((CH, LANES), jnp.float32),
        pltpu.VMEM((CH, LANES), jnp.float32),
        pltpu.VMEM((CH, LANES), jnp.float32),
        pltpu.VMEM((CH, LANES), jnp.float32),
        pltpu.SemaphoreType.DMA,
        pltpu.SemaphoreType.DMA,
    ],
)

BLK = 2048


def _mlp_body(uid, sid, ug, sg, f, w1u4, w1s4, w1f, b1, w2, b2, w3t, b3, out):
    lane = lax.broadcasted_iota(jnp.int32, (BLK, LANES), 1) // EMB
    ku, _ = _section_and_row(uid[...], S_U)
    ks, _ = _section_and_row(sid[...], S_S)
    um = (lane == ku).astype(jnp.float32)
    sm = (lane == ks).astype(jnp.float32)
    h = jnp.dot(ug[...] * um, w1u4[...], preferred_element_type=jnp.float32)
    h += jnp.dot(sg[...] * sm, w1s4[...], preferred_element_type=jnp.float32)
    h += jnp.dot(f[...], w1f[...], preferred_element_type=jnp.float32)
    h = jnp.maximum(h + b1[...], 0.0)
    h2 = jnp.dot(h, w2[...], preferred_element_type=jnp.float32)
    h2 = jnp.maximum(h2 + b2[...], 0.0)
    o = jnp.sum(h2 * w3t[...], axis=1) + b3[0, 0]
    out[...] = 1.0 / (1.0 + jnp.exp(-o))


def _rows(i):
    return (i, 0)


@jax.jit
def kernel(user_id, store_id, sentiment, rating, distance, hour_sin,
           user_table, store_table, W1, b1, W2, b2, W3, b3):
    uid = user_id.astype(jnp.int32)
    sid = store_id.astype(jnp.int32)
    ut = _repack(user_table.T, NU, S_U)
    st = _repack(store_table.T, NST, S_S)
    ug, sg = _sc_gather(ut, st,
                        uid.reshape(NW, NCHUNK, CH), sid.reshape(NW, NCHUNK, CH))

    f = jnp.stack([sentiment, rating, distance, hour_sin], axis=1)
    w1u4 = jnp.concatenate([W1[:EMB]] * PACK, axis=0)
    w1s4 = jnp.concatenate([W1[EMB:2 * EMB]] * PACK, axis=0)
    w1f = W1[2 * EMB:]

    full = lambda shape: pl.BlockSpec(shape, lambda i: (0, 0))
    out = pl.pallas_call(
        _mlp_body,
        grid=(B // BLK,),
        in_specs=[
            pl.BlockSpec((BLK, 1), _rows),
            pl.BlockSpec((BLK, 1), _rows),
            pl.BlockSpec((BLK, LANES), _rows),
            pl.BlockSpec((BLK, LANES), _rows),
            pl.BlockSpec((BLK, 4), _rows),
            full((LANES, 64)),
            full((LANES, 64)),
            full((4, 64)),
            full((1, 64)),
            full((64, 32)),
            full((1, 32)),
            full((1, 32)),
            full((1, 1)),
        ],
        out_specs=pl.BlockSpec((BLK,), lambda i: (i,)),
        out_shape=jax.ShapeDtypeStruct((B,), jnp.float32),
    )(uid.reshape(B, 1), sid.reshape(B, 1), ug, sg, f, w1u4, w1s4, w1f,
      b1.reshape(1, 64), W2, b2.reshape(1, 32), W3.reshape(1, 32),
      b3.reshape(1, 1))
    return out


# speedup vs baseline: 1.5001x; 1.0009x over previous
import functools

import jax
import jax.numpy as jnp
from jax import lax
from jax.experimental import pallas as pl
from jax.experimental.pallas import tpu as pltpu
from jax.experimental.pallas import tpu_sc as plsc

B = 16384
EMB = 32
PACK = 4
LANES = EMB * PACK
NU = 1000000
NST = 100000
TBLK = 1024
S_U = 250880
S_S = 25600
NC = 2
NS = 16
NW = NC * NS
BPW = B // NW
CH = 128
NCHUNK = BPW // CH
NSLOT = 4
NCH_TOT = 2 * NCHUNK


def _repack_body(x0, x1, x2, x3, eye, out):
    del eye
    for k, xb in enumerate((x0, x1, x2, x3)):
        out[:, k * EMB:(k + 1) * EMB] = jnp.swapaxes(xb[...], 0, 1)


def _repack(t32, n_rows, stride):
    nb = stride // TBLK
    nb_max = -(-n_rows // TBLK) - 1

    def spec(k):
        return pl.BlockSpec(
            (EMB, TBLK), lambda i, k=k: (0, jnp.minimum(i + nb * k, nb_max)))

    eye = jnp.eye(EMB, dtype=jnp.float32)
    return pl.pallas_call(
        _repack_body,
        grid=(nb,),
        in_specs=[spec(0), spec(1), spec(2), spec(3),
                  pl.BlockSpec((EMB, EMB), lambda i: (0, 0))],
        out_specs=pl.BlockSpec((TBLK, LANES), lambda i: (i, 0)),
        out_shape=jax.ShapeDtypeStruct((stride, LANES), jnp.float32),
    )(t32, t32, t32, t32, eye)


def _section_and_row(v, stride):
    one = jnp.int32(1)
    zero = jnp.int32(0)
    k = jnp.where(v >= stride, one, zero)
    k += jnp.where(v >= 2 * stride, one, zero)
    k += jnp.where(v >= 3 * stride, one, zero)
    return k, v - k * jnp.int32(stride)


def _gather_body(user_tab, store_tab, uid, sid, ug_out, sg_out,
                 uidx_v, sidx_v, buf0, buf1, buf2, buf3, gsem, wsem):
    bufs = [buf0, buf1, buf2, buf3]
    wid = lax.axis_index("s") * NC + lax.axis_index("c")
    base = wid * BPW
    pltpu.sync_copy(uid.at[wid], uidx_v)
    pltpu.sync_copy(sid.at[wid], sidx_v)
    for j in range(NCHUNK):
        for t in range(CH // 16):
            s = pl.ds(t * 16, 16)
            _, qu = _section_and_row(uidx_v[j, s], S_U)
            uidx_v[j, s] = qu
            _, qs = _section_and_row(sidx_v[j, s], S_S)
            sidx_v[j, s] = qs

    def chunk_src(ci):
        t, j = ci % 2, ci // 2
        idxv = uidx_v if t == 0 else sidx_v
        tab = user_tab if t == 0 else store_tab
        return tab.at[idxv.at[j]]

    def chunk_dst(ci):
        t, j = ci % 2, ci // 2
        out = ug_out if t == 0 else sg_out
        return out.at[pl.ds(base + j * CH, CH)]

    gh = {}
    wh = {}
    for ci in range(NCH_TOT + 3):
        if ci < NCH_TOT:
            if ci >= NSLOT:
                wh[ci - NSLOT].wait()
            gh[ci] = pltpu.async_copy(chunk_src(ci), bufs[ci % NSLOT], gsem)
        wi = ci - 3
        if 0 <= wi < NCH_TOT:
            gh[wi].wait()
            wh[wi] = pltpu.async_copy(bufs[wi % NSLOT], chunk_dst(wi), wsem)
    for wi in range(NCH_TOT - 3, NCH_TOT):
        wh[wi].wait()


_sc_gather = pl.kernel(
    _gather_body,
    out_type=(
        jax.ShapeDtypeStruct((B, LANES), jnp.float32),
        jax.ShapeDtypeStruct((B, LANES), jnp.float32),
    ),
    mesh=plsc.VectorSubcoreMesh(core_axis_name="c", subcore_axis_name="s"),
    scratch_types=[
        pltpu.VMEM((NCHUNK, CH), jnp.int32),
        pltpu.VMEM((NCHUNK, CH), jnp.int32),
        pltpu.VMEM((CH, LANES), jnp.float32),
        pltpu.VMEM((CH, LANES), jnp.float32),
        pltpu.VMEM((CH, LANES), jnp.float32),
        pltpu.VMEM((CH, LANES), jnp.float32),
        pltpu.SemaphoreType.DMA,
        pltpu.SemaphoreType.DMA,
    ],
)

BLK = 2048


def _mlp_body(uid, sid, ug, sg, f, w1u4, w1s4, w1f, b1, w2, b2, w3t, b3, out):
    lane = lax.broadcasted_iota(jnp.int32, (BLK, LANES), 1) // EMB
    ku, _ = _section_and_row(uid[...], S_U)
    ks, _ = _section_and_row(sid[...], S_S)
    um = (lane == ku).astype(jnp.float32)
    sm = (lane == ks).astype(jnp.float32)
    h = jnp.dot(ug[...] * um, w1u4[...], preferred_element_type=jnp.float32)
    h += jnp.dot(sg[...] * sm, w1s4[...], preferred_element_type=jnp.float32)
    h += jnp.dot(f[...], w1f[...], preferred_element_type=jnp.float32)
    h = jnp.maximum(h + b1[...], 0.0)
    h2 = jnp.dot(h, w2[...], preferred_element_type=jnp.float32)
    h2 = jnp.maximum(h2 + b2[...], 0.0)
    o = jnp.sum(h2 * w3t[...], axis=1) + b3[0, 0]
    out[...] = 1.0 / (1.0 + jnp.exp(-o))


def _rows(i):
    return (i, 0)


@jax.jit
def kernel(user_id, store_id, sentiment, rating, distance, hour_sin,
           user_table, store_table, W1, b1, W2, b2, W3, b3):
    uid = user_id.astype(jnp.int32)
    sid = store_id.astype(jnp.int32)
    ut = _repack(user_table.T, NU, S_U)
    st = _repack(store_table.T, NST, S_S)
    ug, sg = _sc_gather(ut, st,
                        uid.reshape(NW, NCHUNK, CH), sid.reshape(NW, NCHUNK, CH))

    f = jnp.stack([sentiment, rating, distance, hour_sin], axis=1)
    w1u4 = jnp.concatenate([W1[:EMB]] * PACK, axis=0)
    w1s4 = jnp.concatenate([W1[EMB:2 * EMB]] * PACK, axis=0)
    w1f = W1[2 * EMB:]

    full = lambda shape: pl.BlockSpec(shape, lambda i: (0, 0))
    out = pl.pallas_call(
        _mlp_body,
        grid=(B // BLK,),
        in_specs=[
            pl.BlockSpec((BLK, 1), _rows),
            pl.BlockSpec((BLK, 1), _rows),
            pl.BlockSpec((BLK, LANES), _rows),
            pl.BlockSpec((BLK, LANES), _rows),
            pl.BlockSpec((BLK, 4), _rows),
            full((LANES, 64)),
            full((LANES, 64)),
            full((4, 64)),
            full((1, 64)),
            full((64, 32)),
            full((1, 32)),
            full((1, 32)),
            full((1, 1)),
        ],
        out_specs=pl.BlockSpec((BLK,), lambda i: (i,)),
        out_shape=jax.ShapeDtypeStruct((B,), jnp.float32),
    )(uid.reshape(B, 1), sid.reshape(B, 1), ug, sg, f, w1u4, w1s4, w1f,
      b1.reshape(1, 64), W2, b2.reshape(1, 32), W3.reshape(1, 32),
      b3.reshape(1, 1))
    return out


# speedup vs baseline: 1.9526x; 1.3017x over previous
import functools

import jax
import jax.numpy as jnp
from jax import lax
from jax.experimental import pallas as pl
from jax.experimental.pallas import tpu as pltpu
from jax.experimental.pallas import tpu_sc as plsc

B = 16384
EMB = 32
PACK = 4
LANES = EMB * PACK
NU = 1000000
NST = 100000
TBLK = 1024
S_U = 250880
S_S = 25600
NC = 2
NS = 16
NW = NC * NS
BPW = B // NW
CH = 128
NCHUNK = BPW // CH
NSLOT = 4
NCH_TOT = 2 * NCHUNK


def _repack_body(x0, x1, x2, x3, eye, out):
    z = jnp.concatenate([x0[...], x1[...], x2[...], x3[...]], axis=0)
    zh = z.astype(jnp.bfloat16)
    zl = (z - zh.astype(jnp.float32)).astype(jnp.bfloat16)
    dims = (((0,), (0,)), ((), ()))
    yh = lax.dot_general(zh, eye[...], dims,
                         preferred_element_type=jnp.float32)
    yl = lax.dot_general(zl, eye[...], dims,
                         preferred_element_type=jnp.float32)
    out[...] = yh + yl


def _repack(t32, n_rows, stride):
    nb = stride // TBLK
    nb_max = -(-n_rows // TBLK) - 1

    def spec(k):
        return pl.BlockSpec(
            (EMB, TBLK), lambda i, k=k: (0, jnp.minimum(i + nb * k, nb_max)))

    eye = jnp.eye(LANES, dtype=jnp.bfloat16)
    return pl.pallas_call(
        _repack_body,
        grid=(nb,),
        in_specs=[spec(0), spec(1), spec(2), spec(3),
                  pl.BlockSpec((LANES, LANES), lambda i: (0, 0))],
        out_specs=pl.BlockSpec((TBLK, LANES), lambda i: (i, 0)),
        out_shape=jax.ShapeDtypeStruct((stride, LANES), jnp.float32),
    )(t32, t32, t32, t32, eye)


def _section_and_row(v, stride):
    one = jnp.int32(1)
    zero = jnp.int32(0)
    k = jnp.where(v >= stride, one, zero)
    k += jnp.where(v >= 2 * stride, one, zero)
    k += jnp.where(v >= 3 * stride, one, zero)
    return k, v - k * jnp.int32(stride)


def _gather_body(user_tab, store_tab, uid, sid, ug_out, sg_out,
                 uidx_v, sidx_v, buf0, buf1, buf2, buf3, gsem, wsem):
    bufs = [buf0, buf1, buf2, buf3]
    wid = lax.axis_index("s") * NC + lax.axis_index("c")
    base = wid * BPW
    pltpu.sync_copy(uid.at[wid], uidx_v)
    pltpu.sync_copy(sid.at[wid], sidx_v)
    for j in range(NCHUNK):
        for t in range(CH // 16):
            s = pl.ds(t * 16, 16)
            _, qu = _section_and_row(uidx_v[j, s], S_U)
            uidx_v[j, s] = qu
            _, qs = _section_and_row(sidx_v[j, s], S_S)
            sidx_v[j, s] = qs

    def chunk_src(ci):
        t, j = ci % 2, ci // 2
        idxv = uidx_v if t == 0 else sidx_v
        tab = user_tab if t == 0 else store_tab
        return tab.at[idxv.at[j]]

    def chunk_dst(ci):
        t, j = ci % 2, ci // 2
        out = ug_out if t == 0 else sg_out
        return out.at[pl.ds(base + j * CH, CH)]

    gh = {}
    wh = {}
    for ci in range(NCH_TOT + 3):
        if ci < NCH_TOT:
            if ci >= NSLOT:
                wh[ci - NSLOT].wait()
            gh[ci] = pltpu.async_copy(chunk_src(ci), bufs[ci % NSLOT], gsem)
        wi = ci - 3
        if 0 <= wi < NCH_TOT:
            gh[wi].wait()
            wh[wi] = pltpu.async_copy(bufs[wi % NSLOT], chunk_dst(wi), wsem)
    for wi in range(NCH_TOT - 3, NCH_TOT):
        wh[wi].wait()


_sc_gather = pl.kernel(
    _gather_body,
    out_type=(
        jax.ShapeDtypeStruct((B, LANES), jnp.float32),
        jax.ShapeDtypeStruct((B, LANES), jnp.float32),
    ),
    mesh=plsc.VectorSubcoreMesh(core_axis_name="c", subcore_axis_name="s"),
    scratch_types=[
        pltpu.VMEM((NCHUNK, CH), jnp.int32),
        pltpu.VMEM((NCHUNK, CH), jnp.int32),
        pltpu.VMEM((CH, LANES), jnp.float32),
        pltpu.VMEM((CH, LANES), jnp.float32),
        pltpu.VMEM((CH, LANES), jnp.float32),
        pltpu.VMEM((CH, LANES), jnp.float32),
        pltpu.SemaphoreType.DMA,
        pltpu.SemaphoreType.DMA,
    ],
)

BLK = 2048


def _mlp_body(uid, sid, ug, sg, f, w1u4, w1s4, w1f, b1, w2, b2, w3t, b3, out):
    lane = lax.broadcasted_iota(jnp.int32, (BLK, LANES), 1) // EMB
    ku, _ = _section_and_row(uid[...], S_U)
    ks, _ = _section_and_row(sid[...], S_S)
    um = (lane == ku).astype(jnp.float32)
    sm = (lane == ks).astype(jnp.float32)
    h = jnp.dot(ug[...] * um, w1u4[...], preferred_element_type=jnp.float32)
    h += jnp.dot(sg[...] * sm, w1s4[...], preferred_element_type=jnp.float32)
    h += jnp.dot(f[...], w1f[...], preferred_element_type=jnp.float32)
    h = jnp.maximum(h + b1[...], 0.0)
    h2 = jnp.dot(h, w2[...], preferred_element_type=jnp.float32)
    h2 = jnp.maximum(h2 + b2[...], 0.0)
    o = jnp.sum(h2 * w3t[...], axis=1) + b3[0, 0]
    out[...] = 1.0 / (1.0 + jnp.exp(-o))


def _rows(i):
    return (i, 0)


@jax.jit
def kernel(user_id, store_id, sentiment, rating, distance, hour_sin,
           user_table, store_table, W1, b1, W2, b2, W3, b3):
    uid = user_id.astype(jnp.int32)
    sid = store_id.astype(jnp.int32)
    ut = _repack(user_table.T, NU, S_U)
    st = _repack(store_table.T, NST, S_S)
    ug, sg = _sc_gather(ut, st,
                        uid.reshape(NW, NCHUNK, CH), sid.reshape(NW, NCHUNK, CH))

    f = jnp.stack([sentiment, rating, distance, hour_sin], axis=1)
    w1u4 = jnp.concatenate([W1[:EMB]] * PACK, axis=0)
    w1s4 = jnp.concatenate([W1[EMB:2 * EMB]] * PACK, axis=0)
    w1f = W1[2 * EMB:]

    full = lambda shape: pl.BlockSpec(shape, lambda i: (0, 0))
    out = pl.pallas_call(
        _mlp_body,
        grid=(B // BLK,),
        in_specs=[
            pl.BlockSpec((BLK, 1), _rows),
            pl.BlockSpec((BLK, 1), _rows),
            pl.BlockSpec((BLK, LANES), _rows),
            pl.BlockSpec((BLK, LANES), _rows),
            pl.BlockSpec((BLK, 4), _rows),
            full((LANES, 64)),
            full((LANES, 64)),
            full((4, 64)),
            full((1, 64)),
            full((64, 32)),
            full((1, 32)),
            full((1, 32)),
            full((1, 1)),
        ],
        out_specs=pl.BlockSpec((BLK,), lambda i: (i,)),
        out_shape=jax.ShapeDtypeStruct((B,), jnp.float32),
    )(uid.reshape(B, 1), sid.reshape(B, 1), ug, sg, f, w1u4, w1s4, w1f,
      b1.reshape(1, 64), W2, b2.reshape(1, 32), W3.reshape(1, 32),
      b3.reshape(1, 1))
    return out


# speedup vs baseline: 3.2054x; 1.6416x over previous
import functools

import jax
import jax.numpy as jnp
from jax import lax
from jax.experimental import pallas as pl
from jax.experimental.pallas import tpu as pltpu
from jax.experimental.pallas import tpu_sc as plsc

B = 16384
EMB = 32
PACK = 4
LANES = EMB * PACK
NU = 1000000
NST = 100000
TBLK = 4096
S_U = 253952
S_S = 28672
NC = 2
NS = 16
NW = NC * NS
BPW = B // NW
CH = 128
NCHUNK = BPW // CH
NSLOT = 4
NCH_TOT = 2 * NCHUNK


def _repack_body(x0, x1, x2, x3, eye, out):
    z = jnp.concatenate([x0[...], x1[...], x2[...], x3[...]], axis=0)
    zh = z.astype(jnp.bfloat16)
    zl = (z - zh.astype(jnp.float32)).astype(jnp.bfloat16)
    dims = (((0,), (0,)), ((), ()))
    yh = lax.dot_general(zh, eye[...], dims,
                         preferred_element_type=jnp.float32)
    yl = lax.dot_general(zl, eye[...], dims,
                         preferred_element_type=jnp.float32)
    out[...] = yh + yl


def _repack(t32, n_rows, stride):
    nb = stride // TBLK
    nb_max = -(-n_rows // TBLK) - 1

    def spec(k):
        return pl.BlockSpec(
            (EMB, TBLK), lambda i, k=k: (0, jnp.minimum(i + nb * k, nb_max)))

    eye = jnp.eye(LANES, dtype=jnp.bfloat16)
    return pl.pallas_call(
        _repack_body,
        grid=(nb,),
        in_specs=[spec(0), spec(1), spec(2), spec(3),
                  pl.BlockSpec((LANES, LANES), lambda i: (0, 0))],
        out_specs=pl.BlockSpec((TBLK, LANES), lambda i: (i, 0)),
        out_shape=jax.ShapeDtypeStruct((stride, LANES), jnp.float32),
    )(t32, t32, t32, t32, eye)


def _section_and_row(v, stride):
    one = jnp.int32(1)
    zero = jnp.int32(0)
    k = jnp.where(v >= stride, one, zero)
    k += jnp.where(v >= 2 * stride, one, zero)
    k += jnp.where(v >= 3 * stride, one, zero)
    return k, v - k * jnp.int32(stride)


def _make_gather(stride):
    def body(tab, idx, out, idx_v, buf0, buf1, buf2, buf3, gsem, wsem):
        bufs = [buf0, buf1, buf2, buf3]
        wid = lax.axis_index("s") * NC + lax.axis_index("c")
        base = wid * BPW
        pltpu.sync_copy(idx.at[wid], idx_v)
        for j in range(NCHUNK):
            for t in range(CH // 16):
                s = pl.ds(t * 16, 16)
                _, q = _section_and_row(idx_v[j, s], stride)
                idx_v[j, s] = q
        gh = [pltpu.async_copy(tab.at[idx_v.at[j]], bufs[j], gsem)
              for j in range(NCHUNK)]
        wh = []
        for j in range(NCHUNK):
            gh[j].wait()
            wh.append(pltpu.async_copy(
                bufs[j], out.at[pl.ds(base + j * CH, CH)], wsem))
        for w in wh:
            w.wait()

    return pl.kernel(
        body,
        out_type=jax.ShapeDtypeStruct((B, LANES), jnp.float32),
        mesh=plsc.VectorSubcoreMesh(core_axis_name="c", subcore_axis_name="s"),
        scratch_types=[
            pltpu.VMEM((NCHUNK, CH), jnp.int32),
            pltpu.VMEM((CH, LANES), jnp.float32),
            pltpu.VMEM((CH, LANES), jnp.float32),
            pltpu.VMEM((CH, LANES), jnp.float32),
            pltpu.VMEM((CH, LANES), jnp.float32),
            pltpu.SemaphoreType.DMA,
            pltpu.SemaphoreType.DMA,
        ],
    )


_gather_user = _make_gather(S_U)
_gather_store = _make_gather(S_S)

BLK = 4096


def _mlp_body(uid, sid, ug, sg, f, w1u4, w1s4, w1f, b1, w2, b2, w3t, b3, out):
    lane = lax.broadcasted_iota(jnp.int32, (BLK, LANES), 1) // EMB
    ku, _ = _section_and_row(uid[...], S_U)
    ks, _ = _section_and_row(sid[...], S_S)
    um = (lane == ku).astype(jnp.float32)
    sm = (lane == ks).astype(jnp.float32)
    h = jnp.dot(ug[...] * um, w1u4[...], preferred_element_type=jnp.float32)
    h += jnp.dot(sg[...] * sm, w1s4[...], preferred_element_type=jnp.float32)
    h += jnp.dot(f[...], w1f[...], preferred_element_type=jnp.float32)
    h = jnp.maximum(h + b1[...], 0.0)
    h2 = jnp.dot(h, w2[...], preferred_element_type=jnp.float32)
    h2 = jnp.maximum(h2 + b2[...], 0.0)
    o = jnp.sum(h2 * w3t[...], axis=1) + b3[0, 0]
    out[...] = 1.0 / (1.0 + jnp.exp(-o))


def _rows(i):
    return (i, 0)


@jax.jit
def kernel(user_id, store_id, sentiment, rating, distance, hour_sin,
           user_table, store_table, W1, b1, W2, b2, W3, b3):
    uid = user_id.astype(jnp.int32)
    sid = store_id.astype(jnp.int32)
    st = _repack(store_table.T, NST, S_S)
    sg = _gather_store(st, sid.reshape(NW, NCHUNK, CH))
    ut = _repack(user_table.T, NU, S_U)
    ug = _gather_user(ut, uid.reshape(NW, NCHUNK, CH))

    f = jnp.stack([sentiment, rating, distance, hour_sin], axis=1)
    w1u4 = jnp.concatenate([W1[:EMB]] * PACK, axis=0)
    w1s4 = jnp.concatenate([W1[EMB:2 * EMB]] * PACK, axis=0)
    w1f = W1[2 * EMB:]

    full = lambda shape: pl.BlockSpec(shape, lambda i: (0, 0))
    out = pl.pallas_call(
        _mlp_body,
        grid=(B // BLK,),
        in_specs=[
            pl.BlockSpec((BLK, 1), _rows),
            pl.BlockSpec((BLK, 1), _rows),
            pl.BlockSpec((BLK, LANES), _rows),
            pl.BlockSpec((BLK, LANES), _rows),
            pl.BlockSpec((BLK, 4), _rows),
            full((LANES, 64)),
            full((LANES, 64)),
            full((4, 64)),
            full((1, 64)),
            full((64, 32)),
            full((1, 32)),
            full((1, 32)),
            full((1, 1)),
        ],
        out_specs=pl.BlockSpec((BLK,), lambda i: (i,)),
        out_shape=jax.ShapeDtypeStruct((B,), jnp.float32),
    )(uid.reshape(B, 1), sid.reshape(B, 1), ug, sg, f, w1u4, w1s4, w1f,
      b1.reshape(1, 64), W2, b2.reshape(1, 32), W3.reshape(1, 32),
      b3.reshape(1, 1))
    return out


# speedup vs baseline: 3.2508x; 1.0142x over previous
import functools

import jax
import jax.numpy as jnp
from jax import lax
from jax.experimental import pallas as pl
from jax.experimental.pallas import tpu as pltpu
from jax.experimental.pallas import tpu_sc as plsc

B = 16384
EMB = 32
PACK = 4
LANES = EMB * PACK
NU = 1000000
NST = 100000
TBLK = 8192
S_U = 253952
S_S = 32768
NC = 2
NS = 16
NW = NC * NS
BPW = B // NW
CH = 128
NCHUNK = BPW // CH
NSLOT = 4
NCH_TOT = 2 * NCHUNK


def _repack_body(x0, x1, x2, x3, eye, out):
    z = jnp.concatenate([x0[...], x1[...], x2[...], x3[...]], axis=0)
    zh = z.astype(jnp.bfloat16)
    zl = (z - zh.astype(jnp.float32)).astype(jnp.bfloat16)
    dims = (((0,), (0,)), ((), ()))
    yh = lax.dot_general(zh, eye[...], dims,
                         preferred_element_type=jnp.float32)
    yl = lax.dot_general(zl, eye[...], dims,
                         preferred_element_type=jnp.float32)
    out[...] = yh + yl


def _repack(t32, n_rows, stride):
    nb = stride // TBLK
    nb_max = -(-n_rows // TBLK) - 1

    def spec(k):
        return pl.BlockSpec(
            (EMB, TBLK), lambda i, k=k: (0, jnp.minimum(i + nb * k, nb_max)))

    eye = jnp.eye(LANES, dtype=jnp.bfloat16)
    return pl.pallas_call(
        _repack_body,
        grid=(nb,),
        in_specs=[spec(0), spec(1), spec(2), spec(3),
                  pl.BlockSpec((LANES, LANES), lambda i: (0, 0))],
        out_specs=pl.BlockSpec((TBLK, LANES), lambda i: (i, 0)),
        out_shape=jax.ShapeDtypeStruct((stride, LANES), jnp.float32),
    )(t32, t32, t32, t32, eye)


def _section_and_row(v, stride):
    one = jnp.int32(1)
    zero = jnp.int32(0)
    k = jnp.where(v >= stride, one, zero)
    k += jnp.where(v >= 2 * stride, one, zero)
    k += jnp.where(v >= 3 * stride, one, zero)
    return k, v - k * jnp.int32(stride)


def _make_gather(stride):
    def body(tab, idx, out, idx_v, buf0, buf1, buf2, buf3, gsem, wsem):
        bufs = [buf0, buf1, buf2, buf3]
        wid = lax.axis_index("s") * NC + lax.axis_index("c")
        base = wid * BPW
        pltpu.sync_copy(idx.at[wid], idx_v)
        for j in range(NCHUNK):
            for t in range(CH // 16):
                s = pl.ds(t * 16, 16)
                _, q = _section_and_row(idx_v[j, s], stride)
                idx_v[j, s] = q
        gh = [pltpu.async_copy(tab.at[idx_v.at[j]], bufs[j], gsem)
              for j in range(NCHUNK)]
        wh = []
        for j in range(NCHUNK):
            gh[j].wait()
            wh.append(pltpu.async_copy(
                bufs[j], out.at[pl.ds(base + j * CH, CH)], wsem))
        for w in wh:
            w.wait()

    return pl.kernel(
        body,
        out_type=jax.ShapeDtypeStruct((B, LANES), jnp.float32),
        mesh=plsc.VectorSubcoreMesh(core_axis_name="c", subcore_axis_name="s"),
        scratch_types=[
            pltpu.VMEM((NCHUNK, CH), jnp.int32),
            pltpu.VMEM((CH, LANES), jnp.float32),
            pltpu.VMEM((CH, LANES), jnp.float32),
            pltpu.VMEM((CH, LANES), jnp.float32),
            pltpu.VMEM((CH, LANES), jnp.float32),
            pltpu.SemaphoreType.DMA,
            pltpu.SemaphoreType.DMA,
        ],
    )


_gather_user = _make_gather(S_U)
_gather_store = _make_gather(S_S)

BLK = 4096


def _dot3(a, b):
    ah = a.astype(jnp.bfloat16)
    al = (a - ah.astype(jnp.float32)).astype(jnp.bfloat16)
    bh = b.astype(jnp.bfloat16)
    bl = (b - bh.astype(jnp.float32)).astype(jnp.bfloat16)
    pet = jnp.float32
    return (jnp.dot(ah, bh, preferred_element_type=pet) +
            (jnp.dot(ah, bl, preferred_element_type=pet) +
             jnp.dot(al, bh, preferred_element_type=pet)))


def _mlp_body(uid, sid, ug, sg, f, w1u4, w1s4, w1f, b1, w2, b2, w3t, b3, out):
    lane = lax.broadcasted_iota(jnp.int32, (BLK, LANES), 1) // EMB
    ku, _ = _section_and_row(uid[...], S_U)
    ks, _ = _section_and_row(sid[...], S_S)
    um = (lane == ku).astype(jnp.float32)
    sm = (lane == ks).astype(jnp.float32)
    h = _dot3(ug[...] * um, w1u4[...])
    h += _dot3(sg[...] * sm, w1s4[...])
    h += jnp.dot(f[...], w1f[...], preferred_element_type=jnp.float32)
    h = jnp.maximum(h + b1[...], 0.0)
    h2 = _dot3(h, w2[...])
    h2 = jnp.maximum(h2 + b2[...], 0.0)
    o = jnp.sum(h2 * w3t[...], axis=1) + b3[0, 0]
    out[...] = 1.0 / (1.0 + jnp.exp(-o))


def _rows(i):
    return (i, 0)


@jax.jit
def kernel(user_id, store_id, sentiment, rating, distance, hour_sin,
           user_table, store_table, W1, b1, W2, b2, W3, b3):
    uid = user_id.astype(jnp.int32)
    sid = store_id.astype(jnp.int32)
    st = _repack(store_table.T, NST, S_S)
    sg = _gather_store(st, sid.reshape(NW, NCHUNK, CH))
    ut = _repack(user_table.T, NU, S_U)
    ug = _gather_user(ut, uid.reshape(NW, NCHUNK, CH))

    f = jnp.stack([sentiment, rating, distance, hour_sin], axis=1)
    w1u4 = jnp.concatenate([W1[:EMB]] * PACK, axis=0)
    w1s4 = jnp.concatenate([W1[EMB:2 * EMB]] * PACK, axis=0)
    w1f = W1[2 * EMB:]

    full = lambda shape: pl.BlockSpec(shape, lambda i: (0, 0))
    out = pl.pallas_call(
        _mlp_body,
        grid=(B // BLK,),
        in_specs=[
            pl.BlockSpec((BLK, 1), _rows),
            pl.BlockSpec((BLK, 1), _rows),
            pl.BlockSpec((BLK, LANES), _rows),
            pl.BlockSpec((BLK, LANES), _rows),
            pl.BlockSpec((BLK, 4), _rows),
            full((LANES, 64)),
            full((LANES, 64)),
            full((4, 64)),
            full((1, 64)),
            full((64, 32)),
            full((1, 32)),
            full((1, 32)),
            full((1, 1)),
        ],
        out_specs=pl.BlockSpec((BLK,), lambda i: (i,)),
        out_shape=jax.ShapeDtypeStruct((B,), jnp.float32),
    )(uid.reshape(B, 1), sid.reshape(B, 1), ug, sg, f, w1u4, w1s4, w1f,
      b1.reshape(1, 64), W2, b2.reshape(1, 32), W3.reshape(1, 32),
      b3.reshape(1, 1))
    return out


# speedup vs baseline: 3.5844x; 1.1026x over previous
import functools

import jax
import jax.numpy as jnp
from jax import lax
from jax.experimental import pallas as pl
from jax.experimental.pallas import tpu as pltpu
from jax.experimental.pallas import tpu_sc as plsc

B = 16384
EMB = 32
PACK = 4
LANES = EMB * PACK
NU = 1000000
NST = 100000
TBLK = 8192
S_U = 253952
S_S = 32768
NC = 2
NS = 16
NW = NC * NS
BPW = B // NW
CH = 128
NCHUNK = BPW // CH
NSLOT = 4
NCH_TOT = 2 * NCHUNK


def _repack_body(x0, x1, x2, x3, eye, out):
    z = jnp.concatenate([x0[...], x1[...], x2[...], x3[...]], axis=0)
    zh = z.astype(jnp.bfloat16)
    zl = (z - zh.astype(jnp.float32)).astype(jnp.bfloat16)
    dims = (((0,), (0,)), ((), ()))
    yh = lax.dot_general(zh, eye[...], dims,
                         preferred_element_type=jnp.float32)
    yl = lax.dot_general(zl, eye[...], dims,
                         preferred_element_type=jnp.float32)
    out[...] = yh + yl


def _repack(t32, n_rows, stride):
    nb = stride // TBLK
    nb_max = -(-n_rows // TBLK) - 1

    def spec(k):
        return pl.BlockSpec(
            (EMB, TBLK), lambda i, k=k: (0, jnp.minimum(i + nb * k, nb_max)))

    eye = jnp.eye(LANES, dtype=jnp.bfloat16)
    return pl.pallas_call(
        _repack_body,
        grid=(nb,),
        in_specs=[spec(0), spec(1), spec(2), spec(3),
                  pl.BlockSpec((LANES, LANES), lambda i: (0, 0))],
        out_specs=pl.BlockSpec((TBLK, LANES), lambda i: (i, 0)),
        out_shape=jax.ShapeDtypeStruct((stride, LANES), jnp.float32),
    )(t32, t32, t32, t32, eye)


def _section_and_row(v, stride):
    one = jnp.int32(1)
    zero = jnp.int32(0)
    k = jnp.where(v >= stride, one, zero)
    k += jnp.where(v >= 2 * stride, one, zero)
    k += jnp.where(v >= 3 * stride, one, zero)
    return k, v - k * jnp.int32(stride)


def _make_gather(stride):
    def body(tab, idx, out, idx_v, buf0, buf1, buf2, buf3, gsem, wsem):
        bufs = [buf0, buf1, buf2, buf3]
        wid = lax.axis_index("s") * NC + lax.axis_index("c")
        base = wid * BPW
        pltpu.sync_copy(idx.at[wid], idx_v)
        for j in range(NCHUNK):
            for t in range(CH // 16):
                s = pl.ds(t * 16, 16)
                _, q = _section_and_row(idx_v[j, s], stride)
                idx_v[j, s] = q
        gh = [pltpu.async_copy(tab.at[idx_v.at[j]], bufs[j], gsem)
              for j in range(NCHUNK)]
        wh = []
        for j in range(NCHUNK):
            gh[j].wait()
            wh.append(pltpu.async_copy(
                bufs[j], out.at[pl.ds(base + j * CH, CH)], wsem))
        for w in wh:
            w.wait()

    return pl.kernel(
        body,
        out_type=jax.ShapeDtypeStruct((B, LANES), jnp.float32),
        mesh=plsc.VectorSubcoreMesh(core_axis_name="c", subcore_axis_name="s"),
        scratch_types=[
            pltpu.VMEM((NCHUNK, CH), jnp.int32),
            pltpu.VMEM((CH, LANES), jnp.float32),
            pltpu.VMEM((CH, LANES), jnp.float32),
            pltpu.VMEM((CH, LANES), jnp.float32),
            pltpu.VMEM((CH, LANES), jnp.float32),
            pltpu.SemaphoreType.DMA,
            pltpu.SemaphoreType.DMA,
        ],
    )


_gather_user = _make_gather(S_U)
_gather_store = _make_gather(S_S)

BLK = 4096


def _dot3(a, b):
    ah = a.astype(jnp.bfloat16)
    al = (a - ah.astype(jnp.float32)).astype(jnp.bfloat16)
    bh = b.astype(jnp.bfloat16)
    bl = (b - bh.astype(jnp.float32)).astype(jnp.bfloat16)
    pet = jnp.float32
    return (jnp.dot(ah, bh, preferred_element_type=pet) +
            (jnp.dot(ah, bl, preferred_element_type=pet) +
             jnp.dot(al, bh, preferred_element_type=pet)))


def _mlp_body(uid, sid, ug, sg, f, w1u4, w1s4, w1f, b1, w2, b2, w3t, b3, out):
    lane = lax.broadcasted_iota(jnp.int32, (BLK, LANES), 1) // EMB
    ku, _ = _section_and_row(uid[...], S_U)
    ks, _ = _section_and_row(sid[...], S_S)
    um = (lane == ku).astype(jnp.float32)
    sm = (lane == ks).astype(jnp.float32)
    h = jnp.dot(ug[...] * um, w1u4[...], preferred_element_type=jnp.float32)
    h += jnp.dot(sg[...] * sm, w1s4[...], preferred_element_type=jnp.float32)
    h += jnp.dot(f[...], w1f[...], preferred_element_type=jnp.float32)
    h = jnp.maximum(h + b1[...], 0.0)
    h2 = jnp.dot(h, w2[...], preferred_element_type=jnp.float32)
    h2 = jnp.maximum(h2 + b2[...], 0.0)
    o = jnp.sum(h2 * w3t[...], axis=1) + b3[0, 0]
    out[...] = 1.0 / (1.0 + jnp.exp(-o))


def _rows(i):
    return (i, 0)


@jax.jit
def kernel(user_id, store_id, sentiment, rating, distance, hour_sin,
           user_table, store_table, W1, b1, W2, b2, W3, b3):
    uid = user_id.astype(jnp.int32)
    sid = store_id.astype(jnp.int32)
    st = _repack(store_table.T, NST, S_S)
    sg = _gather_store(st, sid.reshape(NW, NCHUNK, CH))
    user_t, _ = lax.optimization_barrier((user_table, st))
    ut = _repack(user_t.T, NU, S_U)
    ug = _gather_user(ut, uid.reshape(NW, NCHUNK, CH))

    f = jnp.stack([sentiment, rating, distance, hour_sin], axis=1)
    w1u4 = jnp.concatenate([W1[:EMB]] * PACK, axis=0)
    w1s4 = jnp.concatenate([W1[EMB:2 * EMB]] * PACK, axis=0)
    w1f = W1[2 * EMB:]

    full = lambda shape: pl.BlockSpec(shape, lambda i: (0, 0))
    out = pl.pallas_call(
        _mlp_body,
        grid=(B // BLK,),
        in_specs=[
            pl.BlockSpec((BLK, 1), _rows),
            pl.BlockSpec((BLK, 1), _rows),
            pl.BlockSpec((BLK, LANES), _rows),
            pl.BlockSpec((BLK, LANES), _rows),
            pl.BlockSpec((BLK, 4), _rows),
            full((LANES, 64)),
            full((LANES, 64)),
            full((4, 64)),
            full((1, 64)),
            full((64, 32)),
            full((1, 32)),
            full((1, 32)),
            full((1, 1)),
        ],
        out_specs=pl.BlockSpec((BLK,), lambda i: (i,)),
        out_shape=jax.ShapeDtypeStruct((B,), jnp.float32),
    )(uid.reshape(B, 1), sid.reshape(B, 1), ug, sg, f, w1u4, w1s4, w1f,
      b1.reshape(1, 64), W2, b2.reshape(1, 32), W3.reshape(1, 32),
      b3.reshape(1, 1))
    return out
